# baseline (speedup 1.0000x reference)
import functools
import math

import jax
import jax.numpy as jnp
from jax import lax
from jax.experimental import pallas as pl
from jax.experimental.pallas import tpu as pltpu
from jax.experimental.pallas import tpu_sc as plsc

_MAP_SIZE = 1000000
_D = 64
_PAD = 0
_BATCH = 4096
_HIST = 200
_B = _BATCH * _HIST
_SCALE = math.sqrt(_D)

_NW = 32
_CHUNK = 512
_SUB = _CHUNK // 128
_BW = _B // _NW
_NCH = _BW // _CHUNK
_IDX_ROWS = _B // 128


def _emb_body(idx_hbm, table_hbm, out_hbm, idx_v, rows_v, sem):
    wid = lax.axis_index("s") * 2 + lax.axis_index("c")
    base = wid * _BW
    irow0 = wid * (_BW // 128)

    def chunk_body(g, carry):
        pltpu.sync_copy(idx_hbm.at[pl.ds(irow0 + g * _SUB, _SUB)], idx_v)

        handles = []
        for k in range(_SUB):
            handles.append(
                pltpu.async_copy(
                    table_hbm.at[idx_v.at[k]],
                    rows_v.at[pl.ds(k * 128, 128)],
                    sem,
                )
            )
        for h in handles:
            h.wait()

        def scale_group(i, c2):
            k = i // (128 // 16)
            o = (i % (128 // 16)) * 16
            idx16 = idx_v[k, pl.ds(o, 16)]
            s16 = jnp.where(idx16 == _PAD, 0.0, _SCALE)
            for lane in range(16):
                s = s16[lane]
                r = i * 16 + lane
                for c in range(_D // 16):
                    sl = pl.ds(c * 16, 16)
                    rows_v[r, sl] = rows_v[r, sl] * s
            return c2

        lax.fori_loop(0, _CHUNK // 16, scale_group, 0)

        pltpu.sync_copy(rows_v, out_hbm.at[pl.ds(base + g * _CHUNK, _CHUNK)])
        return carry

    lax.fori_loop(0, _NCH, chunk_body, 0)


_mesh = plsc.VectorSubcoreMesh(core_axis_name="c", subcore_axis_name="s")

_emb = functools.partial(
    pl.kernel,
    mesh=_mesh,
    out_type=jax.ShapeDtypeStruct((_B, _D), jnp.float32),
    compiler_params=pltpu.CompilerParams(use_tc_tiling_on_sc=False),
    scratch_types=[
        pltpu.VMEM((_SUB, 128), jnp.int32),
        pltpu.VMEM((_CHUNK, _D), jnp.float32),
        pltpu.SemaphoreType.DMA,
    ],
)(_emb_body)


@jax.jit
def kernel(x, table):
    idx = x.reshape(_IDX_ROWS, 128)
    out = _emb(idx, table)
    return out.reshape(_BATCH, _HIST, _D)


# speedup vs baseline: 1.1882x; 1.1882x over previous
import functools
import math

import jax
import jax.numpy as jnp
from jax import lax
from jax.experimental import pallas as pl
from jax.experimental.pallas import tpu as pltpu
from jax.experimental.pallas import tpu_sc as plsc

_D = 64
_PAD = 0
_BATCH = 4096
_HIST = 200
_B = _BATCH * _HIST
_SCALE = math.sqrt(_D)

_NW = 32
_CHUNK = 512
_SUB = _CHUNK // 128
_NB = 2
_BW = _B // _NW
_NCH = _BW // _CHUNK
_IDX_ROWS = _B // 128


def _emb_body(idx_hbm, table_hbm, out_hbm, idx_v, rows_v, gsem, osem):
    wid = lax.axis_index("s") * 2 + lax.axis_index("c")
    base = wid * _BW
    irow0 = wid * (_BW // 128)

    def copy_idx(g, slot):
        pltpu.sync_copy(idx_hbm.at[pl.ds(irow0 + g * _SUB, _SUB)], idx_v.at[slot])

    def fire_gathers(slot):
        for k in range(_SUB):
            pltpu.async_copy(
                table_hbm.at[idx_v.at[slot, k]],
                rows_v.at[slot, pl.ds(k * 128, 128)],
                gsem[slot],
            )

    def wait_gathers(slot):
        for k in range(_SUB):
            pltpu.make_async_copy(
                table_hbm.at[idx_v.at[slot, k]],
                rows_v.at[slot, pl.ds(k * 128, 128)],
                gsem[slot],
            ).wait()

    def fire_out(g, slot):
        pltpu.async_copy(
            rows_v.at[slot],
            out_hbm.at[pl.ds(base + g * _CHUNK, _CHUNK)],
            osem[slot],
        )

    def wait_out(g, slot):
        pltpu.make_async_copy(
            rows_v.at[slot],
            out_hbm.at[pl.ds(base + g * _CHUNK, _CHUNK)],
            osem[slot],
        ).wait()

    def scale(slot):
        def scale_group(i, c2):
            k = i // (128 // 16)
            o = (i % (128 // 16)) * 16
            idx16 = idx_v[slot, k, pl.ds(o, 16)]
            s16 = jnp.where(idx16 == _PAD, 0.0, _SCALE)
            for lane in range(16):
                s = s16[lane]
                r = i * 16 + lane
                for c in range(_D // 16):
                    sl = pl.ds(c * 16, 16)
                    rows_v[slot, r, sl] = rows_v[slot, r, sl] * s
            return c2

        lax.fori_loop(0, _CHUNK // 16, scale_group, 0)

    copy_idx(0, 0)
    fire_gathers(0)

    def outer(t, carry):
        for b in range(_NB):
            g = t * _NB + b
            slot = b
            other = (b + 1) % _NB
            nxt = g + 1

            @pl.when(nxt < _NCH)
            def _():
                @pl.when(nxt >= _NB)
                def _():
                    wait_out(nxt - _NB, other)

                copy_idx(nxt, other)
                fire_gathers(other)

            wait_gathers(slot)
            scale(slot)
            fire_out(g, slot)
        return carry

    lax.fori_loop(0, _NCH // _NB, outer, 0)

    for j in range(_NCH - _NB, _NCH):
        wait_out(j, j % _NB)


_mesh = plsc.VectorSubcoreMesh(core_axis_name="c", subcore_axis_name="s")

_emb = functools.partial(
    pl.kernel,
    mesh=_mesh,
    out_type=jax.ShapeDtypeStruct((_B, _D), jnp.float32),
    compiler_params=pltpu.CompilerParams(use_tc_tiling_on_sc=False),
    scratch_types=[
        pltpu.VMEM((_NB, _SUB, 128), jnp.int32),
        pltpu.VMEM((_NB, _CHUNK, _D), jnp.float32),
        [pltpu.SemaphoreType.DMA] * _NB,
        [pltpu.SemaphoreType.DMA] * _NB,
    ],
)(_emb_body)


@jax.jit
def kernel(x, table):
    idx = x.reshape(_IDX_ROWS, 128)
    out = _emb(idx, table)
    return out.reshape(_BATCH, _HIST, _D)


# speedup vs baseline: 1.3446x; 1.1317x over previous
import functools
import math

import jax
import jax.numpy as jnp
from jax import lax
from jax.experimental import pallas as pl
from jax.experimental.pallas import tpu as pltpu
from jax.experimental.pallas import tpu_sc as plsc

_D = 64
_PAD = 0
_BATCH = 4096
_HIST = 200
_SCALE = math.sqrt(_D)

_NW = 32
_ROWS_W = _BATCH // _NW
_NBR = 4
_NB = 2
_NCH = _ROWS_W // _NBR
_G0 = 128
_G1 = _HIST - _G0


def _emb_body(x_hbm, table_hbm, out_hbm, idx_v, rows_v, gsem, osem):
    wid = lax.axis_index("s") * 2 + lax.axis_index("c")
    row0 = wid * _ROWS_W

    def copy_idx(g, slot):
        pltpu.sync_copy(x_hbm.at[pl.ds(row0 + g * _NBR, _NBR)], idx_v.at[slot])

    def gather_descs(slot):
        descs = []
        for r in range(_NBR):
            descs.append((
                table_hbm.at[idx_v.at[slot, r, pl.ds(0, _G0)]],
                rows_v.at[slot, r, pl.ds(0, _G0)],
            ))
            descs.append((
                table_hbm.at[idx_v.at[slot, r, pl.ds(_G0, _G1)]],
                rows_v.at[slot, r, pl.ds(_G0, _G1)],
            ))
        return descs

    def fire_gathers(slot):
        for src, dst in gather_descs(slot):
            pltpu.async_copy(src, dst, gsem[slot])

    def wait_gathers(slot):
        for src, dst in gather_descs(slot):
            pltpu.make_async_copy(src, dst, gsem[slot]).wait()

    def fire_out(g, slot):
        pltpu.async_copy(
            rows_v.at[slot],
            out_hbm.at[pl.ds(row0 + g * _NBR, _NBR)],
            osem[slot],
        )

    def wait_out(g, slot):
        pltpu.make_async_copy(
            rows_v.at[slot],
            out_hbm.at[pl.ds(row0 + g * _NBR, _NBR)],
            osem[slot],
        ).wait()

    def scale(slot):
        def scale_row(r, c2):
            for o, lane0 in [(i * 16, 0) for i in range(12)] + [(184, 8)]:
                idx16 = idx_v[slot, r, pl.ds(o, 16)]
                s16 = jnp.where(idx16 == _PAD, 0.0, _SCALE)
                for lane in range(lane0, 16):
                    s = s16[lane]
                    for c in range(_D // 16):
                        sl = pl.ds(c * 16, 16)
                        rows_v[slot, r, o + lane, sl] = (
                            rows_v[slot, r, o + lane, sl] * s
                        )
            return c2

        lax.fori_loop(0, _NBR, scale_row, 0)

    copy_idx(0, 0)
    fire_gathers(0)

    def outer(t, carry):
        for b in range(_NB):
            g = t * _NB + b
            slot = b
            other = (b + 1) % _NB
            nxt = g + 1

            @pl.when(nxt < _NCH)
            def _():
                @pl.when(nxt >= _NB)
                def _():
                    wait_out(nxt - _NB, other)

                copy_idx(nxt, other)
                fire_gathers(other)

            wait_gathers(slot)
            scale(slot)
            fire_out(g, slot)
        return carry

    lax.fori_loop(0, _NCH // _NB, outer, 0)

    for j in range(_NCH - _NB, _NCH):
        wait_out(j, j % _NB)


_mesh = plsc.VectorSubcoreMesh(core_axis_name="c", subcore_axis_name="s")

_emb = functools.partial(
    pl.kernel,
    mesh=_mesh,
    out_type=jax.ShapeDtypeStruct((_BATCH, _HIST, _D), jnp.float32),
    compiler_params=pltpu.CompilerParams(use_tc_tiling_on_sc=False),
    scratch_types=[
        pltpu.VMEM((_NB, _NBR, _HIST), jnp.int32),
        pltpu.VMEM((_NB, _NBR, _HIST, _D), jnp.float32),
        [pltpu.SemaphoreType.DMA] * _NB,
        [pltpu.SemaphoreType.DMA] * _NB,
    ],
)(_emb_body)


@jax.jit
def kernel(x, table):
    return _emb(x, table)
